# baseline (speedup 1.0000x reference)
import dataclasses
import functools

import jax
import jax.numpy as jnp
from jax import lax
from jax.experimental import pallas as pl
from jax.experimental.pallas import tpu as pltpu
from jax.experimental.pallas import tpu_sc as plsc

NC = 2
NS = 16
NT = NC * NS
LANES = 16
KCH = 2560
CLIST = 16384


def _mesh():
    return plsc.VectorSubcoreMesh(core_axis_name="c", subcore_axis_name="s")


def _cp():
    cp = pltpu.CompilerParams()
    if "needs_layout_passes" in pltpu.CompilerParams.__dataclass_fields__:
        cp = dataclasses.replace(cp, needs_layout_passes=False)
    return cp


def _sc_degree(dst_ch, n, rpt):
    nch = dst_ch.shape[0]
    kch = dst_ch.shape[2]
    ngrp = kch // LANES

    @functools.partial(
        pl.kernel,
        out_type=jax.ShapeDtypeStruct((NC, NS, rpt), jnp.float32),
        mesh=_mesh(),
        compiler_params=_cp(),
        scratch_types=[
            pltpu.VMEM((1, kch), jnp.int32),
            pltpu.VMEM((rpt,), jnp.float32),
        ],
    )
    def deg_kernel(dst_hbm, out_hbm, didx_v, deg_v):
        c = lax.axis_index("c")
        s = lax.axis_index("s")
        w = c * NS + s
        lo = w * rpt
        ones16 = jnp.ones((LANES,), jnp.float32)

        @pl.loop(0, rpt // LANES)
        def _(i):
            deg_v[pl.ds(i * LANES, LANES)] = jnp.zeros((LANES,), jnp.float32)

        @pl.loop(0, nch)
        def _(j):
            pltpu.sync_copy(dst_hbm.at[j], didx_v)

            @pl.loop(0, ngrp)
            def _(g):
                dstg = didx_v[0, pl.ds(g * LANES, LANES)]
                ldst = dstg - lo
                m = (ldst >= 0) & (ldst < rpt)
                ldst = jnp.minimum(jnp.maximum(ldst, 0), rpt - 1)
                plsc.addupdate_scatter(deg_v, [ldst], ones16, mask=m)

        pltpu.sync_copy(deg_v, out_hbm.at[c, s])

    return deg_kernel(dst_ch)


def _sc_aggregate(hs, src_ch, dst_ch, rpt):
    n, d = hs.shape
    nch = src_ch.shape[0]
    kch = src_ch.shape[2]
    ngrp = kch // LANES

    @functools.partial(
        pl.kernel,
        out_type=jax.ShapeDtypeStruct((NC, NS, rpt + 1, d), jnp.float32),
        mesh=_mesh(),
        compiler_params=_cp(),
        scratch_types=[
            pltpu.VMEM((1, kch), jnp.int32),
            pltpu.VMEM((1, kch), jnp.int32),
            pltpu.VMEM((CLIST,), jnp.int32),
            pltpu.VMEM((CLIST,), jnp.int32),
            pltpu.VMEM((LANES, d), jnp.float32),
            pltpu.VMEM((rpt + 1, d), jnp.float32),
        ],
    )
    def agg_kernel(hs_hbm, src_hbm, dst_hbm, out_hbm, sidx_v, didx_v,
                   cls_v, cld_v, rows_v, acc_v):
        c = lax.axis_index("c")
        s = lax.axis_index("s")
        w = c * NS + s
        lo = w * rpt
        iota = lax.broadcasted_iota(jnp.int32, (LANES,), 0)
        zrow = jnp.zeros((LANES,), jnp.float32)

        @pl.loop(0, rpt + 1)
        def _(i):
            @pl.loop(0, d // LANES)
            def _(jj):
                acc_v[i, pl.ds(jj * LANES, LANES)] = zrow

        def chunk_body(j, p):
            pltpu.sync_copy(src_hbm.at[j], sidx_v)
            pltpu.sync_copy(dst_hbm.at[j], didx_v)

            def grp_body(g, p):
                dstg = didx_v[0, pl.ds(g * LANES, LANES)]
                srcg = sidx_v[0, pl.ds(g * LANES, LANES)]
                ldst = dstg - lo
                m = (ldst >= 0) & (ldst < rpt)
                cs = plsc.cumsum(m.astype(jnp.int32))
                pos = jnp.maximum(p + cs - 1, 0)
                wm = m & (pos < CLIST)
                plsc.store_scatter(cls_v, [pos], srcg, mask=wm)
                ldst = jnp.minimum(jnp.maximum(ldst, 0), rpt - 1)
                plsc.store_scatter(cld_v, [pos], ldst, mask=wm)
                return p + jnp.max(cs)

            return pl.loop(0, ngrp, init_carry=p)(grp_body)

        p = pl.loop(0, nch, init_carry=jnp.int32(0))(chunk_body)

        tpos = jnp.minimum(p + iota, CLIST - 1)
        plsc.store_scatter(cls_v, [tpos], jnp.zeros((LANES,), jnp.int32),
                           mask=tpos >= p)
        plsc.store_scatter(cld_v, [tpos],
                           jnp.full((LANES,), rpt, jnp.int32), mask=tpos >= p)
        ng = jnp.minimum((p + LANES - 1) // LANES, CLIST // LANES)

        @pl.loop(0, ng)
        def _(t):
            sg = cls_v[pl.ds(t * LANES, LANES)]
            dg = cld_v[pl.ds(t * LANES, LANES)]
            pltpu.sync_copy(hs_hbm.at[sg], rows_v)
            for c0 in range(d):
                col = jnp.full((LANES,), c0, jnp.int32)
                vals = plsc.load_gather(rows_v, [iota, col])
                plsc.addupdate_scatter(acc_v, [dg, col], vals)

        pltpu.sync_copy(acc_v, out_hbm.at[c, s])

    return agg_kernel(hs, src_ch, dst_ch)


_DOT = functools.partial(
    lax.dot_general,
    dimension_numbers=(((1,), (0,)), ((), ())),
    preferred_element_type=jnp.float32,
    precision=lax.Precision.HIGHEST,
)


def _tc_matmul(x, w):
    n, din = x.shape
    dout = w.shape[1]
    bm = 1000

    def mm_kernel(x_ref, w_ref, o_ref):
        o_ref[...] = _DOT(x_ref[...], w_ref[...])

    return pl.pallas_call(
        mm_kernel,
        grid=(n // bm,),
        in_specs=[
            pl.BlockSpec((bm, din), lambda i: (i, 0)),
            pl.BlockSpec((din, dout), lambda i: (0, 0)),
        ],
        out_specs=pl.BlockSpec((bm, dout), lambda i: (i, 0)),
        out_shape=jax.ShapeDtypeStruct((n, dout), jnp.float32),
    )(x, w)


def _tc_scale(h, deg):
    n, d = h.shape
    bm = 1000

    def k(h_ref, g_ref, hs_ref, dinv_ref):
        dinv = lax.rsqrt(g_ref[...] + 1.0)
        dinv_ref[...] = dinv
        hs_ref[...] = h_ref[...] * dinv

    return pl.pallas_call(
        k,
        grid=(n // bm,),
        in_specs=[
            pl.BlockSpec((bm, d), lambda i: (i, 0)),
            pl.BlockSpec((bm, 1), lambda i: (i, 0)),
        ],
        out_specs=[
            pl.BlockSpec((bm, d), lambda i: (i, 0)),
            pl.BlockSpec((bm, 1), lambda i: (i, 0)),
        ],
        out_shape=[
            jax.ShapeDtypeStruct((n, d), jnp.float32),
            jax.ShapeDtypeStruct((n, 1), jnp.float32),
        ],
    )(h, deg)


def _tc_mid(acc, hs1, dinv, b1, w2):
    n, d = hs1.shape
    bm = 1000

    def k(a_ref, h_ref, d_ref, b_ref, w_ref, o_ref):
        dv = d_ref[...]
        z = (a_ref[...] + h_ref[...]) * dv + b_ref[...]
        o_ref[...] = _DOT(jnp.maximum(z, 0.0), w_ref[...]) * dv

    return pl.pallas_call(
        k,
        grid=(n // bm,),
        in_specs=[
            pl.BlockSpec((bm, d), lambda i: (i, 0)),
            pl.BlockSpec((bm, d), lambda i: (i, 0)),
            pl.BlockSpec((bm, 1), lambda i: (i, 0)),
            pl.BlockSpec((1, d), lambda i: (0, 0)),
            pl.BlockSpec((d, d), lambda i: (0, 0)),
        ],
        out_specs=pl.BlockSpec((bm, d), lambda i: (i, 0)),
        out_shape=jax.ShapeDtypeStruct((n, d), jnp.float32),
    )(acc, hs1, dinv, b1, w2)


def _tc_final(acc, hs2, dinv, b2):
    n, d = hs2.shape
    bm = 1000

    def k(a_ref, h_ref, d_ref, b_ref, o_ref):
        z = (a_ref[...] + h_ref[...]) * d_ref[...] + b_ref[...]
        m = jnp.max(z, axis=1, keepdims=True)
        lse = jnp.log(jnp.sum(jnp.exp(z - m), axis=1, keepdims=True)) + m
        o_ref[...] = z - lse

    return pl.pallas_call(
        k,
        grid=(n // bm,),
        in_specs=[
            pl.BlockSpec((bm, d), lambda i: (i, 0)),
            pl.BlockSpec((bm, d), lambda i: (i, 0)),
            pl.BlockSpec((bm, 1), lambda i: (i, 0)),
            pl.BlockSpec((1, d), lambda i: (0, 0)),
        ],
        out_specs=pl.BlockSpec((bm, d), lambda i: (i, 0)),
        out_shape=jax.ShapeDtypeStruct((n, d), jnp.float32),
    )(acc, hs2, dinv, b2)


def kernel(x, edge_index, W1, b1, W2, b2):
    n, d = x.shape
    e = edge_index.shape[1]
    rpt = -(-n // NT)
    rpt = -(-rpt // LANES) * LANES
    nch = e // KCH
    src_ch = edge_index[0].reshape(nch, 1, KCH)
    dst_ch = edge_index[1].reshape(nch, 1, KCH)

    hist = _sc_degree(dst_ch, n, rpt)
    deg = hist.reshape(NT * rpt)[:n].reshape(n, 1)
    h1 = _tc_matmul(x, W1)
    hs1, dinv = _tc_scale(h1, deg)

    acc1 = _sc_aggregate(hs1, src_ch, dst_ch, rpt)
    acc1 = acc1.reshape(NT, rpt + 1, d)[:, :rpt].reshape(NT * rpt, d)[:n]
    hs2 = _tc_mid(acc1, hs1, dinv, b1.reshape(1, -1), W2)

    acc2 = _sc_aggregate(hs2, src_ch, dst_ch, rpt)
    acc2 = acc2.reshape(NT, rpt + 1, d)[:, :rpt].reshape(NT * rpt, d)[:n]
    return _tc_final(acc2, hs2, dinv, b2.reshape(1, -1))


# speedup vs baseline: 1.1692x; 1.1692x over previous
import dataclasses
import functools

import jax
import jax.numpy as jnp
from jax import lax
from jax.experimental import pallas as pl
from jax.experimental.pallas import tpu as pltpu
from jax.experimental.pallas import tpu_sc as plsc

NC = 2
NS = 16
NT = NC * NS
LANES = 16
KCH = 2560
CLIST = 16384


def _mesh():
    return plsc.VectorSubcoreMesh(core_axis_name="c", subcore_axis_name="s")


def _cp():
    cp = pltpu.CompilerParams()
    if "needs_layout_passes" in pltpu.CompilerParams.__dataclass_fields__:
        cp = dataclasses.replace(cp, needs_layout_passes=False)
    return cp


def _sc_degree(dst_ch, n, rpt):
    nch = dst_ch.shape[0]
    kch = dst_ch.shape[2]
    ngrp = kch // LANES

    @functools.partial(
        pl.kernel,
        out_type=jax.ShapeDtypeStruct((NC, NS, rpt), jnp.float32),
        mesh=_mesh(),
        compiler_params=_cp(),
        scratch_types=[
            pltpu.VMEM((1, kch), jnp.int32),
            pltpu.VMEM((rpt,), jnp.float32),
        ],
    )
    def deg_kernel(dst_hbm, out_hbm, didx_v, deg_v):
        c = lax.axis_index("c")
        s = lax.axis_index("s")
        w = c * NS + s
        lo = w * rpt
        ones16 = jnp.ones((LANES,), jnp.float32)

        @pl.loop(0, rpt // LANES)
        def _(i):
            deg_v[pl.ds(i * LANES, LANES)] = jnp.zeros((LANES,), jnp.float32)

        @pl.loop(0, nch)
        def _(j):
            pltpu.sync_copy(dst_hbm.at[j], didx_v)

            @pl.loop(0, ngrp)
            def _(g):
                dstg = didx_v[0, pl.ds(g * LANES, LANES)]
                ldst = dstg - lo
                m = (ldst >= 0) & (ldst < rpt)
                ldst = jnp.minimum(jnp.maximum(ldst, 0), rpt - 1)
                plsc.addupdate_scatter(deg_v, [ldst], ones16, mask=m)

        pltpu.sync_copy(deg_v, out_hbm.at[c, s])

    return deg_kernel(dst_ch)


def _sc_aggregate(hs, src_ch, dst_ch, rpt):
    n, d = hs.shape
    nch = src_ch.shape[0]
    kch = src_ch.shape[2]
    ngrp = kch // LANES

    @functools.partial(
        pl.kernel,
        out_type=jax.ShapeDtypeStruct((NC, NS, rpt + 1, d), jnp.float32),
        mesh=_mesh(),
        compiler_params=_cp(),
        scratch_types=[
            pltpu.VMEM((1, kch), jnp.int32),
            pltpu.VMEM((1, kch), jnp.int32),
            pltpu.VMEM((CLIST,), jnp.int32),
            pltpu.VMEM((CLIST,), jnp.int32),
            pltpu.VMEM((2, LANES, d), jnp.float32),
            pltpu.VMEM((rpt + 1, d), jnp.float32),
            pltpu.SemaphoreType.DMA,
        ],
    )
    def agg_kernel(hs_hbm, src_hbm, dst_hbm, out_hbm, sidx_v, didx_v,
                   cls_v, cld_v, rows_v, acc_v, gsem):
        c = lax.axis_index("c")
        s = lax.axis_index("s")
        w = c * NS + s
        lo = w * rpt
        iota = lax.broadcasted_iota(jnp.int32, (LANES,), 0)
        zrow = jnp.zeros((LANES,), jnp.float32)

        @pl.loop(0, rpt + 1)
        def _(i):
            @pl.loop(0, d // LANES)
            def _(jj):
                acc_v[i, pl.ds(jj * LANES, LANES)] = zrow

        def chunk_body(j, p):
            pltpu.sync_copy(src_hbm.at[j], sidx_v)
            pltpu.sync_copy(dst_hbm.at[j], didx_v)

            def grp_body(g, p):
                dstg = didx_v[0, pl.ds(g * LANES, LANES)]
                srcg = sidx_v[0, pl.ds(g * LANES, LANES)]
                ldst = dstg - lo
                m = (ldst >= 0) & (ldst < rpt)
                cs = plsc.cumsum(m.astype(jnp.int32))
                pos = jnp.maximum(p + cs - 1, 0)
                wm = m & (pos < CLIST)
                plsc.store_scatter(cls_v, [pos], srcg, mask=wm)
                ldst = jnp.minimum(jnp.maximum(ldst, 0), rpt - 1)
                plsc.store_scatter(cld_v, [pos], ldst, mask=wm)
                return p + jnp.max(cs)

            return pl.loop(0, ngrp, init_carry=p)(grp_body)

        p = pl.loop(0, nch, init_carry=jnp.int32(0))(chunk_body)

        tpos = jnp.minimum(p + iota, CLIST - 1)
        plsc.store_scatter(cls_v, [tpos], jnp.zeros((LANES,), jnp.int32),
                           mask=tpos >= p)
        plsc.store_scatter(cld_v, [tpos],
                           jnp.full((LANES,), rpt, jnp.int32), mask=tpos >= p)
        ng = jnp.minimum((p + LANES - 1) // LANES, CLIST // LANES)

        @pl.when(ng > 0)
        def _():
            sg0 = cls_v[pl.ds(0, LANES)]
            pltpu.async_copy(hs_hbm.at[sg0], rows_v.at[0], gsem)

        @pl.loop(0, ng)
        def _(t):
            par = jnp.bitwise_and(t, 1)
            sg = cls_v[pl.ds(t * LANES, LANES)]
            dg = cld_v[pl.ds(t * LANES, LANES)]
            pltpu.make_async_copy(hs_hbm.at[sg], rows_v.at[par], gsem).wait()

            @pl.when(t + 1 < ng)
            def _():
                sg1 = cls_v[pl.ds((t + 1) * LANES, LANES)]
                pltpu.async_copy(
                    hs_hbm.at[sg1], rows_v.at[jnp.bitwise_and(t + 1, 1)], gsem
                )

            parv = jnp.zeros((LANES,), jnp.int32) + par
            for c0 in range(d):
                col = jnp.full((LANES,), c0, jnp.int32)
                vals = plsc.load_gather(rows_v, [parv, iota, col])
                plsc.addupdate_scatter(acc_v, [dg, col], vals)

        pltpu.sync_copy(acc_v, out_hbm.at[c, s])

    return agg_kernel(hs, src_ch, dst_ch)


_DOT = functools.partial(
    lax.dot_general,
    dimension_numbers=(((1,), (0,)), ((), ())),
    preferred_element_type=jnp.float32,
    precision=lax.Precision.HIGHEST,
)


def _tc_matmul(x, w):
    n, din = x.shape
    dout = w.shape[1]
    bm = 1000

    def mm_kernel(x_ref, w_ref, o_ref):
        o_ref[...] = _DOT(x_ref[...], w_ref[...])

    return pl.pallas_call(
        mm_kernel,
        grid=(n // bm,),
        in_specs=[
            pl.BlockSpec((bm, din), lambda i: (i, 0)),
            pl.BlockSpec((din, dout), lambda i: (0, 0)),
        ],
        out_specs=pl.BlockSpec((bm, dout), lambda i: (i, 0)),
        out_shape=jax.ShapeDtypeStruct((n, dout), jnp.float32),
    )(x, w)


def _tc_scale(h, deg):
    n, d = h.shape
    bm = 1000

    def k(h_ref, g_ref, hs_ref, dinv_ref):
        dinv = lax.rsqrt(g_ref[...] + 1.0)
        dinv_ref[...] = dinv
        hs_ref[...] = h_ref[...] * dinv

    return pl.pallas_call(
        k,
        grid=(n // bm,),
        in_specs=[
            pl.BlockSpec((bm, d), lambda i: (i, 0)),
            pl.BlockSpec((bm, 1), lambda i: (i, 0)),
        ],
        out_specs=[
            pl.BlockSpec((bm, d), lambda i: (i, 0)),
            pl.BlockSpec((bm, 1), lambda i: (i, 0)),
        ],
        out_shape=[
            jax.ShapeDtypeStruct((n, d), jnp.float32),
            jax.ShapeDtypeStruct((n, 1), jnp.float32),
        ],
    )(h, deg)


def _tc_mid(acc, hs1, dinv, b1, w2):
    n, d = hs1.shape
    bm = 1000

    def k(a_ref, h_ref, d_ref, b_ref, w_ref, o_ref):
        dv = d_ref[...]
        z = (a_ref[...] + h_ref[...]) * dv + b_ref[...]
        o_ref[...] = _DOT(jnp.maximum(z, 0.0), w_ref[...]) * dv

    return pl.pallas_call(
        k,
        grid=(n // bm,),
        in_specs=[
            pl.BlockSpec((bm, d), lambda i: (i, 0)),
            pl.BlockSpec((bm, d), lambda i: (i, 0)),
            pl.BlockSpec((bm, 1), lambda i: (i, 0)),
            pl.BlockSpec((1, d), lambda i: (0, 0)),
            pl.BlockSpec((d, d), lambda i: (0, 0)),
        ],
        out_specs=pl.BlockSpec((bm, d), lambda i: (i, 0)),
        out_shape=jax.ShapeDtypeStruct((n, d), jnp.float32),
    )(acc, hs1, dinv, b1, w2)


def _tc_final(acc, hs2, dinv, b2):
    n, d = hs2.shape
    bm = 1000

    def k(a_ref, h_ref, d_ref, b_ref, o_ref):
        z = (a_ref[...] + h_ref[...]) * d_ref[...] + b_ref[...]
        m = jnp.max(z, axis=1, keepdims=True)
        lse = jnp.log(jnp.sum(jnp.exp(z - m), axis=1, keepdims=True)) + m
        o_ref[...] = z - lse

    return pl.pallas_call(
        k,
        grid=(n // bm,),
        in_specs=[
            pl.BlockSpec((bm, d), lambda i: (i, 0)),
            pl.BlockSpec((bm, d), lambda i: (i, 0)),
            pl.BlockSpec((bm, 1), lambda i: (i, 0)),
            pl.BlockSpec((1, d), lambda i: (0, 0)),
        ],
        out_specs=pl.BlockSpec((bm, d), lambda i: (i, 0)),
        out_shape=jax.ShapeDtypeStruct((n, d), jnp.float32),
    )(acc, hs2, dinv, b2)


def kernel(x, edge_index, W1, b1, W2, b2):
    n, d = x.shape
    e = edge_index.shape[1]
    rpt = -(-n // NT)
    rpt = -(-rpt // LANES) * LANES
    nch = e // KCH
    src_ch = edge_index[0].reshape(nch, 1, KCH)
    dst_ch = edge_index[1].reshape(nch, 1, KCH)

    hist = _sc_degree(dst_ch, n, rpt)
    deg = hist.reshape(NT * rpt)[:n].reshape(n, 1)
    h1 = _tc_matmul(x, W1)
    hs1, dinv = _tc_scale(h1, deg)

    acc1 = _sc_aggregate(hs1, src_ch, dst_ch, rpt)
    acc1 = acc1.reshape(NT, rpt + 1, d)[:, :rpt].reshape(NT * rpt, d)[:n]
    hs2 = _tc_mid(acc1, hs1, dinv, b1.reshape(1, -1), W2)

    acc2 = _sc_aggregate(hs2, src_ch, dst_ch, rpt)
    acc2 = acc2.reshape(NT, rpt + 1, d)[:, :rpt].reshape(NT * rpt, d)[:n]
    return _tc_final(acc2, hs2, dinv, b2.reshape(1, -1))


# speedup vs baseline: 1.2556x; 1.0739x over previous
import dataclasses
import functools

import jax
import jax.numpy as jnp
from jax import lax
from jax.experimental import pallas as pl
from jax.experimental.pallas import tpu as pltpu
from jax.experimental.pallas import tpu_sc as plsc

NC = 2
NS = 16
NT = NC * NS
LANES = 16
KCH = 2560
CLIST = 16384


def _mesh():
    return plsc.VectorSubcoreMesh(core_axis_name="c", subcore_axis_name="s")


def _cp():
    cp = pltpu.CompilerParams()
    if "needs_layout_passes" in pltpu.CompilerParams.__dataclass_fields__:
        cp = dataclasses.replace(cp, needs_layout_passes=False)
    return cp


def _sc_degree(dst_ch, n, rpt):
    nch = dst_ch.shape[0]
    kch = dst_ch.shape[2]
    ngrp = kch // LANES

    @functools.partial(
        pl.kernel,
        out_type=jax.ShapeDtypeStruct((NC, NS, rpt), jnp.float32),
        mesh=_mesh(),
        compiler_params=_cp(),
        scratch_types=[
            pltpu.VMEM((1, kch), jnp.int32),
            pltpu.VMEM((rpt,), jnp.float32),
        ],
    )
    def deg_kernel(dst_hbm, out_hbm, didx_v, deg_v):
        c = lax.axis_index("c")
        s = lax.axis_index("s")
        w = c * NS + s
        lo = w * rpt
        ones16 = jnp.ones((LANES,), jnp.float32)

        @pl.loop(0, rpt // LANES)
        def _(i):
            deg_v[pl.ds(i * LANES, LANES)] = jnp.zeros((LANES,), jnp.float32)

        @pl.loop(0, nch)
        def _(j):
            pltpu.sync_copy(dst_hbm.at[j], didx_v)

            @pl.loop(0, ngrp)
            def _(g):
                dstg = didx_v[0, pl.ds(g * LANES, LANES)]
                ldst = dstg - lo
                m = (ldst >= 0) & (ldst < rpt)
                ldst = jnp.minimum(jnp.maximum(ldst, 0), rpt - 1)
                plsc.addupdate_scatter(deg_v, [ldst], ones16, mask=m)

        pltpu.sync_copy(deg_v, out_hbm.at[c, s])

    return deg_kernel(dst_ch)


def _sc_aggregate(hs, src_ch, dst_ch, rpt):
    n, d = hs.shape
    nch = src_ch.shape[0]
    kch = src_ch.shape[2]
    ngrp = kch // LANES

    @functools.partial(
        pl.kernel,
        out_type=jax.ShapeDtypeStruct((NC, NS, rpt + 1, d), jnp.float32),
        mesh=_mesh(),
        compiler_params=_cp(),
        scratch_types=[
            pltpu.VMEM((2, 1, kch), jnp.int32),
            pltpu.VMEM((2, 1, kch), jnp.int32),
            pltpu.VMEM((CLIST,), jnp.int32),
            pltpu.VMEM((CLIST,), jnp.int32),
            pltpu.VMEM((2, LANES, d), jnp.float32),
            pltpu.VMEM((rpt + 1, d), jnp.float32),
            pltpu.SemaphoreType.DMA,
            pltpu.SemaphoreType.DMA,
        ],
    )
    def agg_kernel(hs_hbm, src_hbm, dst_hbm, out_hbm, sidx_v, didx_v,
                   cls_v, cld_v, rows_v, acc_v, gsem, isem):
        c = lax.axis_index("c")
        s = lax.axis_index("s")
        w = c * NS + s
        lo = w * rpt
        iota = lax.broadcasted_iota(jnp.int32, (LANES,), 0)
        zrow = jnp.zeros((LANES,), jnp.float32)

        @pl.loop(0, rpt + 1)
        def _(i):
            @pl.loop(0, d // LANES)
            def _(jj):
                acc_v[i, pl.ds(jj * LANES, LANES)] = zrow

        pltpu.async_copy(src_hbm.at[0], sidx_v.at[0], isem)
        pltpu.async_copy(dst_hbm.at[0], didx_v.at[0], isem)

        def chunk_body(j, p):
            par = jnp.bitwise_and(j, 1)
            pltpu.make_async_copy(src_hbm.at[j], sidx_v.at[par], isem).wait()
            pltpu.make_async_copy(dst_hbm.at[j], didx_v.at[par], isem).wait()

            @pl.when(j + 1 < nch)
            def _():
                npar = jnp.bitwise_and(j + 1, 1)
                pltpu.async_copy(src_hbm.at[j + 1], sidx_v.at[npar], isem)
                pltpu.async_copy(dst_hbm.at[j + 1], didx_v.at[npar], isem)

            def grp_body(g, p):
                dstg = didx_v[par, 0, pl.ds(g * LANES, LANES)]
                srcg = sidx_v[par, 0, pl.ds(g * LANES, LANES)]
                ldst = dstg - lo
                m = (ldst >= 0) & (ldst < rpt)
                cs = plsc.cumsum(m.astype(jnp.int32))
                pos = jnp.maximum(p + cs - 1, 0)
                wm = m & (pos < CLIST)
                plsc.store_scatter(cls_v, [pos], srcg, mask=wm)
                ldst = jnp.minimum(jnp.maximum(ldst, 0), rpt - 1)
                plsc.store_scatter(cld_v, [pos], ldst, mask=wm)
                return p + jnp.max(cs)

            return pl.loop(0, ngrp, init_carry=p)(grp_body)

        p = pl.loop(0, nch, init_carry=jnp.int32(0))(chunk_body)

        tpos = jnp.minimum(p + iota, CLIST - 1)
        plsc.store_scatter(cls_v, [tpos], jnp.zeros((LANES,), jnp.int32),
                           mask=tpos >= p)
        plsc.store_scatter(cld_v, [tpos],
                           jnp.full((LANES,), rpt, jnp.int32), mask=tpos >= p)
        ng = jnp.minimum((p + LANES - 1) // LANES, CLIST // LANES)

        @pl.when(ng > 0)
        def _():
            sg0 = cls_v[pl.ds(0, LANES)]
            pltpu.async_copy(hs_hbm.at[sg0], rows_v.at[0], gsem)

        @pl.loop(0, ng)
        def _(t):
            par = jnp.bitwise_and(t, 1)
            sg = cls_v[pl.ds(t * LANES, LANES)]
            dg = cld_v[pl.ds(t * LANES, LANES)]
            pltpu.make_async_copy(hs_hbm.at[sg], rows_v.at[par], gsem).wait()

            @pl.when(t + 1 < ng)
            def _():
                sg1 = cls_v[pl.ds((t + 1) * LANES, LANES)]
                pltpu.async_copy(
                    hs_hbm.at[sg1], rows_v.at[jnp.bitwise_and(t + 1, 1)], gsem
                )

            parv = jnp.zeros((LANES,), jnp.int32) + par
            for c0 in range(d):
                col = jnp.full((LANES,), c0, jnp.int32)
                vals = plsc.load_gather(rows_v, [parv, iota, col])
                plsc.addupdate_scatter(acc_v, [dg, col], vals)

        pltpu.sync_copy(acc_v, out_hbm.at[c, s])

    return agg_kernel(hs, src_ch, dst_ch)


_DOT = functools.partial(
    lax.dot_general,
    dimension_numbers=(((1,), (0,)), ((), ())),
    preferred_element_type=jnp.float32,
    precision=lax.Precision.HIGHEST,
)


def _tc_matmul(x, w):
    n, din = x.shape
    dout = w.shape[1]
    bm = 1000

    def mm_kernel(x_ref, w_ref, o_ref):
        o_ref[...] = _DOT(x_ref[...], w_ref[...])

    return pl.pallas_call(
        mm_kernel,
        grid=(n // bm,),
        in_specs=[
            pl.BlockSpec((bm, din), lambda i: (i, 0)),
            pl.BlockSpec((din, dout), lambda i: (0, 0)),
        ],
        out_specs=pl.BlockSpec((bm, dout), lambda i: (i, 0)),
        out_shape=jax.ShapeDtypeStruct((n, dout), jnp.float32),
    )(x, w)


def _tc_scale(h, deg):
    n, d = h.shape
    bm = 1000

    def k(h_ref, g_ref, hs_ref, dinv_ref):
        dinv = lax.rsqrt(g_ref[...] + 1.0)
        dinv_ref[...] = dinv
        hs_ref[...] = h_ref[...] * dinv

    return pl.pallas_call(
        k,
        grid=(n // bm,),
        in_specs=[
            pl.BlockSpec((bm, d), lambda i: (i, 0)),
            pl.BlockSpec((bm, 1), lambda i: (i, 0)),
        ],
        out_specs=[
            pl.BlockSpec((bm, d), lambda i: (i, 0)),
            pl.BlockSpec((bm, 1), lambda i: (i, 0)),
        ],
        out_shape=[
            jax.ShapeDtypeStruct((n, d), jnp.float32),
            jax.ShapeDtypeStruct((n, 1), jnp.float32),
        ],
    )(h, deg)


def _tc_mid(acc, hs1, dinv, b1, w2):
    n, d = hs1.shape
    bm = 1000

    def k(a_ref, h_ref, d_ref, b_ref, w_ref, o_ref):
        dv = d_ref[...]
        z = (a_ref[...] + h_ref[...]) * dv + b_ref[...]
        o_ref[...] = _DOT(jnp.maximum(z, 0.0), w_ref[...]) * dv

    return pl.pallas_call(
        k,
        grid=(n // bm,),
        in_specs=[
            pl.BlockSpec((bm, d), lambda i: (i, 0)),
            pl.BlockSpec((bm, d), lambda i: (i, 0)),
            pl.BlockSpec((bm, 1), lambda i: (i, 0)),
            pl.BlockSpec((1, d), lambda i: (0, 0)),
            pl.BlockSpec((d, d), lambda i: (0, 0)),
        ],
        out_specs=pl.BlockSpec((bm, d), lambda i: (i, 0)),
        out_shape=jax.ShapeDtypeStruct((n, d), jnp.float32),
    )(acc, hs1, dinv, b1, w2)


def _tc_final(acc, hs2, dinv, b2):
    n, d = hs2.shape
    bm = 1000

    def k(a_ref, h_ref, d_ref, b_ref, o_ref):
        z = (a_ref[...] + h_ref[...]) * d_ref[...] + b_ref[...]
        m = jnp.max(z, axis=1, keepdims=True)
        lse = jnp.log(jnp.sum(jnp.exp(z - m), axis=1, keepdims=True)) + m
        o_ref[...] = z - lse

    return pl.pallas_call(
        k,
        grid=(n // bm,),
        in_specs=[
            pl.BlockSpec((bm, d), lambda i: (i, 0)),
            pl.BlockSpec((bm, d), lambda i: (i, 0)),
            pl.BlockSpec((bm, 1), lambda i: (i, 0)),
            pl.BlockSpec((1, d), lambda i: (0, 0)),
        ],
        out_specs=pl.BlockSpec((bm, d), lambda i: (i, 0)),
        out_shape=jax.ShapeDtypeStruct((n, d), jnp.float32),
    )(acc, hs2, dinv, b2)


def kernel(x, edge_index, W1, b1, W2, b2):
    n, d = x.shape
    e = edge_index.shape[1]
    rpt = -(-n // NT)
    rpt = -(-rpt // LANES) * LANES
    nch = e // KCH
    src_ch = edge_index[0].reshape(nch, 1, KCH)
    dst_ch = edge_index[1].reshape(nch, 1, KCH)

    hist = _sc_degree(dst_ch, n, rpt)
    deg = hist.reshape(NT * rpt)[:n].reshape(n, 1)
    h1 = _tc_matmul(x, W1)
    hs1, dinv = _tc_scale(h1, deg)

    acc1 = _sc_aggregate(hs1, src_ch, dst_ch, rpt)
    acc1 = acc1.reshape(NT, rpt + 1, d)[:, :rpt].reshape(NT * rpt, d)[:n]
    hs2 = _tc_mid(acc1, hs1, dinv, b1.reshape(1, -1), W2)

    acc2 = _sc_aggregate(hs2, src_ch, dst_ch, rpt)
    acc2 = acc2.reshape(NT, rpt + 1, d)[:, :rpt].reshape(NT * rpt, d)[:n]
    return _tc_final(acc2, hs2, dinv, b2.reshape(1, -1))
